# baseline (speedup 1.0000x reference)
import functools

import jax
import jax.numpy as jnp
from jax import lax
from jax.experimental import pallas as pl
from jax.experimental.pallas import tpu as pltpu
from jax.experimental.pallas import tpu_sc as plsc

_NC = 2
_NS = 16
_L = 16
_NW = _NC * _NS

_MULT_CUR = 36313
_MULT_PREV = 27191
_CHUNK = 128


def _make_sc_kernel(N, V, D, per_w, n_chunks):
    mesh = plsc.VectorSubcoreMesh(core_axis_name="c", subcore_axis_name="s")

    @functools.partial(
        pl.kernel,
        out_type=jax.ShapeDtypeStruct((N, D), jnp.float32),
        mesh=mesh,
        scratch_types=[
            pltpu.VMEM((per_w,), jnp.int32),
            pltpu.VMEM((per_w,), jnp.int32),
            pltpu.VMEM((n_chunks, _CHUNK), jnp.int32),
            pltpu.VMEM((per_w, D), jnp.float32),
            pltpu.VMEM((_L,), jnp.float32),
            pltpu.SemaphoreType.DMA,
            pltpu.SemaphoreType.DMA,
            pltpu.SemaphoreType.DMA,
        ],
        compiler_params=pltpu.CompilerParams(use_tc_tiling_on_sc=False),
    )
    def sc_kernel(t_hbm, p_hbm, s_hbm, embed_hbm, out_hbm,
                  t_v, p_v, idx_v, rows_v, s_v, sem_in, sem_g, sem_out):
        wid = lax.axis_index("s") * _NC + lax.axis_index("c")
        base = wid * per_w

        cp_t = pltpu.async_copy(t_hbm.at[pl.ds(base, per_w)], t_v, sem_in)
        cp_p = pltpu.async_copy(p_hbm.at[pl.ds(base, per_w)], p_v, sem_in)
        pltpu.sync_copy(s_hbm, s_v)
        cp_t.wait()
        cp_p.wait()

        vregs_per_chunk = _CHUNK // _L
        for j in range(per_w // _L):
            cur = t_v[pl.ds(j * _L, _L)]
            prv = p_v[pl.ds(j * _L, _L)]
            h = lax.bitwise_xor(cur * _MULT_CUR, prv * _MULT_PREV) % (V - 1)
            idx_v[j // vregs_per_chunk,
                  pl.ds((j % vregs_per_chunk) * _L, _L)] = h

        gathers = []
        for c in range(n_chunks):
            gathers.append(pltpu.async_copy(
                embed_hbm.at[idx_v.at[c]],
                rows_v.at[pl.ds(c * _CHUNK, _CHUNK)],
                sem_g))

        sv = s_v[...]
        stores = []
        for c in range(n_chunks):
            gathers[c].wait()

            def scale_row(r, _):
                row = c * _CHUNK + r
                for cc in range(D // _L):
                    rows_v[row, pl.ds(cc * _L, _L)] = (
                        rows_v[row, pl.ds(cc * _L, _L)] * sv)
                return 0

            lax.fori_loop(0, _CHUNK, scale_row, 0)
            stores.append(pltpu.async_copy(
                rows_v.at[pl.ds(c * _CHUNK, _CHUNK)],
                out_hbm.at[pl.ds(base + c * _CHUNK, _CHUNK)],
                sem_out))
        for cp in stores:
            cp.wait()

    return sc_kernel


def kernel(x, embed, scale):
    B, S = x.shape
    V, D = embed.shape
    N = B * S
    per_w = N // _NW
    n_chunks = per_w // _CHUNK

    t = x.astype(jnp.int32)
    prev = jnp.concatenate([jnp.zeros_like(t[:, :1]), t[:, :-1]], axis=1)
    scale_vec = jnp.full((_L,), scale, jnp.float32)

    sc = _make_sc_kernel(N, V, D, per_w, n_chunks)
    out = sc(t.reshape(N), prev.reshape(N), scale_vec, embed)
    return out.reshape(B, S, D)


# speedup vs baseline: 1.6838x; 1.6838x over previous
import functools

import jax
import jax.numpy as jnp
from jax import lax
from jax.experimental import pallas as pl
from jax.experimental.pallas import tpu as pltpu
from jax.experimental.pallas import tpu_sc as plsc

_NC = 2
_NS = 16
_L = 16
_NW = _NC * _NS

_MULT_CUR = 36313
_MULT_PREV = 27191


def _make_sc_kernel(N, V, D, per_w):
    mesh = plsc.VectorSubcoreMesh(core_axis_name="c", subcore_axis_name="s")

    @functools.partial(
        pl.kernel,
        out_type=jax.ShapeDtypeStruct((N, D), jnp.float32),
        mesh=mesh,
        scratch_types=[
            pltpu.VMEM((per_w,), jnp.int32),
            pltpu.VMEM((per_w,), jnp.int32),
            pltpu.VMEM((per_w,), jnp.int32),
            pltpu.VMEM((per_w, D), jnp.float32),
            pltpu.VMEM((_L,), jnp.float32),
            pltpu.SemaphoreType.DMA,
            pltpu.SemaphoreType.DMA,
            pltpu.SemaphoreType.DMA,
        ],
    )
    def sc_kernel(t_hbm, p_hbm, s_hbm, embed_hbm, out_hbm,
                  t_v, p_v, idx_v, rows_v, s_v, sem_in, sem_g, sem_out):
        wid = lax.axis_index("s") * _NC + lax.axis_index("c")
        base = wid * per_w

        cp_t = pltpu.async_copy(t_hbm.at[pl.ds(base, per_w)], t_v, sem_in)
        cp_p = pltpu.async_copy(p_hbm.at[pl.ds(base, per_w)], p_v, sem_in)
        pltpu.sync_copy(s_hbm, s_v)
        cp_t.wait()
        cp_p.wait()

        for j in range(per_w // _L):
            cur = t_v[pl.ds(j * _L, _L)]
            prv = p_v[pl.ds(j * _L, _L)]
            h = lax.bitwise_xor(cur * _MULT_CUR, prv * _MULT_PREV) % (V - 1)
            idx_v[pl.ds(j * _L, _L)] = h

        def fire(g, _):
            gbase = g * _L
            hvec = idx_v[pl.ds(gbase, _L)]
            for k in range(_L):
                pltpu.async_copy(
                    embed_hbm.at[hvec[k]], rows_v.at[gbase + k], sem_g)
            return 0

        lax.fori_loop(0, per_w // _L, fire, 0)
        pltpu.make_async_copy(
            out_hbm.at[pl.ds(base, per_w)], rows_v, sem_g).wait()

        sv = s_v[...]

        def scale_row(r, _):
            for cc in range(D // _L):
                rows_v[r, pl.ds(cc * _L, _L)] = (
                    rows_v[r, pl.ds(cc * _L, _L)] * sv)
            return 0

        lax.fori_loop(0, per_w, scale_row, 0)
        pltpu.async_copy(
            rows_v, out_hbm.at[pl.ds(base, per_w)], sem_out).wait()

    return sc_kernel


def kernel(x, embed, scale):
    B, S = x.shape
    V, D = embed.shape
    N = B * S
    per_w = N // _NW

    t = x.astype(jnp.int32)
    prev = jnp.concatenate([jnp.zeros_like(t[:, :1]), t[:, :-1]], axis=1)
    scale_vec = jnp.full((_L,), scale, jnp.float32)

    sc = _make_sc_kernel(N, V, D, per_w)
    out = sc(t.reshape(N), prev.reshape(N), scale_vec, embed)
    return out.reshape(B, S, D)
